# Initial kernel scaffold; baseline (speedup 1.0000x reference)
import jax
import jax.numpy as jnp
from jax.experimental import pallas as pl


def kernel(positions, emb0, emb1, emb2, emb3, emb4, identity_emb, transition_gates, transition_smoothness):
    raise NotImplementedError("write your pallas kernel here")


# speedup vs baseline: 2.5178x; 2.5178x over previous
import functools

import jax
import jax.numpy as jnp
from jax import lax
from jax.experimental import pallas as pl
from jax.experimental.pallas import tpu as pltpu
from jax.experimental.pallas import tpu_sc as plsc

BOUNDS = (0, 5, 55, 605, 6655, 73205, 805254)
D = 1024
L = 16
NC, NS = 2, 16
NW = NC * NS


def _splat(val, n=L, dtype=jnp.int32):
    return jnp.full((n,), val, dtype=dtype)


def _sc_forward(B, positions_flat, e0, e1, e2, e3, e4, identity_emb,
                gates16, inv_smooth16):
    BPW = B // NW
    GROUPS = BPW // L
    LCAP = BPW + L
    tables = (e0, e1, e2, e3, e4)

    mesh = plsc.VectorSubcoreMesh(
        core_axis_name="c", subcore_axis_name="s",
        num_cores=NC, num_subcores=NS)

    @functools.partial(
        pl.kernel,
        out_type=jax.ShapeDtypeStruct((B, D), jnp.float32),
        mesh=mesh,
        compiler_params=pltpu.CompilerParams(needs_layout_passes=False),
        scratch_types=[
            pltpu.VMEM((5 * LCAP,), jnp.int32),
            pltpu.VMEM((5 * LCAP,), jnp.int32),
            pltpu.VMEM((5 * LCAP,), jnp.float32),
            pltpu.VMEM((L, D), jnp.float32),
            pltpu.VMEM((5, D), jnp.float32),
            pltpu.VMEM((L,), jnp.float32),
            pltpu.VMEM((L,), jnp.float32),
            pltpu.VMEM((BPW,), jnp.int32),
            pltpu.SemaphoreType.DMA,
        ],
    )
    def k(pos_hbm, t0, t1, t2, t3, t4, id_hbm, gates_hbm, ismo_hbm, out_hbm,
          loc_v, dst_v, scl_v, gat_v, id_v, gates_v, ismo_v, pos_v, sem):
        tabs = (t0, t1, t2, t3, t4)
        wid = lax.axis_index("s") * NC + lax.axis_index("c")
        base = wid * BPW

        pltpu.sync_copy(pos_hbm.at[pl.ds(base, BPW)], pos_v)
        pltpu.sync_copy(id_hbm, id_v)
        pltpu.sync_copy(gates_hbm, gates_v)
        pltpu.sync_copy(ismo_hbm, ismo_v)

        iota = lax.iota(jnp.int32, L)
        ismo = ismo_v[...]

        def compact_body(g, cnts):
            p = plsc.load_gather(pos_v, [_splat(g * L) + iota])
            pa = jnp.abs(p)
            r = _splat(0)
            start = _splat(0)
            nxt = _splat(BOUNDS[1])
            for kb in range(1, 5):
                above = pa >= BOUNDS[kb]
                r = r + above.astype(jnp.int32)
                start = jnp.where(above, _splat(BOUNDS[kb]), start)
                nxt = jnp.where(above, _splat(BOUNDS[kb + 1]), nxt)
            local = jnp.clip(pa - start, _splat(0), nxt - start - 1)
            dist = (nxt - p).astype(jnp.float32)
            gate = 1.0 / (1.0 + jnp.exp(-dist * ismo))
            rgate = plsc.load_gather(gates_v, [r])
            scale = gate * rgate
            dest = _splat(base + g * L) + iota
            new_cnts = []
            for i in range(5):
                m = r == i
                pc = plsc.cumsum(m.astype(jnp.int32))
                slot = _splat(i * LCAP) + cnts[i] + pc - 1
                plsc.store_scatter(loc_v, [slot], local, mask=m)
                plsc.store_scatter(dst_v, [slot], dest, mask=m)
                plsc.store_scatter(scl_v, [slot], scale, mask=m)
                new_cnts.append(cnts[i] + plsc.all_reduce_population_count(m))
            return tuple(new_cnts)

        cnts = lax.fori_loop(
            0, GROUPS, compact_body, tuple(_splat(0) for _ in range(5)))

        for i in range(5):
            cnt_s = jnp.max(cnts[i])
            lbase = i * LCAP

            @pl.when(cnt_s > 0)
            def _process():
                lastq = _splat(lbase) + cnts[i] - 1
                tail = _splat(lbase) + cnts[i] + iota
                plsc.store_scatter(loc_v, [tail], plsc.load_gather(loc_v, [lastq]))
                plsc.store_scatter(dst_v, [tail], plsc.load_gather(dst_v, [lastq]))
                plsc.store_scatter(scl_v, [tail], plsc.load_gather(scl_v, [lastq]))
                n_g = (cnt_s + (L - 1)) >> 4

                def group_body(t, _):
                    q = _splat(lbase + t * L) + iota
                    lidx = plsc.load_gather(loc_v, [q])
                    pltpu.async_copy(tabs[i].at[lidx], gat_v, sem).wait()

                    def row_body(j, _):
                        sj = plsc.load_gather(scl_v, [_splat(lbase + t * L + j)])
                        jv = _splat(j)
                        iv = _splat(i)

                        def col_body(c, _):
                            col = _splat(c * L) + iota
                            v = plsc.load_gather(gat_v, [jv, col])
                            dv = plsc.load_gather(id_v, [iv, col])
                            plsc.store_scatter(gat_v, [jv, col], (v + dv) * sj)
                            return 0

                        lax.fori_loop(0, D // L, col_body, 0)
                        return 0

                    lax.fori_loop(0, L, row_body, 0)
                    didx = plsc.load_gather(dst_v, [q])
                    pltpu.async_copy(gat_v, out_hbm.at[didx], sem).wait()
                    return 0

                lax.fori_loop(0, n_g, group_body, 0)

    return k(positions_flat, e0, e1, e2, e3, e4, identity_emb,
             gates16, inv_smooth16)


def kernel(positions, emb0, emb1, emb2, emb3, emb4, identity_emb,
           transition_gates, transition_smoothness):
    shape = positions.shape
    B = positions.size
    pos_flat = positions.reshape(B)
    gates16 = jnp.zeros((L,), jnp.float32).at[:5].set(transition_gates)
    inv_smooth16 = jnp.full((L,), 1.0, jnp.float32) / transition_smoothness[0]
    out = _sc_forward(B, pos_flat, emb0, emb1, emb2, emb3, emb4,
                      identity_emb, gates16, inv_smooth16)
    return out.reshape(shape + (D,))
